# baseline (speedup 1.0000x reference)
import functools

import jax
import jax.numpy as jnp
from jax import lax
from jax.experimental import pallas as pl
from jax.experimental.pallas import tpu as pltpu
from jax.experimental.pallas import tpu_sc as plsc

BATCH = 16384
NF = 32

_info = plsc.get_sparse_core_info()
_NC, _NS, _L = _info.num_cores, _info.num_subcores, _info.num_lanes
NW = _NC * _NS
B_PER_W = BATCH // NW
NCHUNK = 4
CH = B_PER_W // NCHUNK

_mesh = plsc.VectorSubcoreMesh(core_axis_name="c", subcore_axis_name="s")


@functools.partial(
    pl.kernel,
    mesh=_mesh,
    out_type=jax.ShapeDtypeStruct((BATCH,), jnp.float32),
    compiler_params=pltpu.CompilerParams(needs_layout_passes=False,
                                         use_tc_tiling_on_sc=False),
    scratch_types=[
        pltpu.VMEM((NCHUNK, CH), jnp.int32),
        pltpu.VMEM((NCHUNK, CH), jnp.int32),
        pltpu.VMEM((CH, NF), jnp.float32),
        pltpu.VMEM((CH, NF), jnp.float32),
        pltpu.VMEM((CH, NF), jnp.float32),
        pltpu.VMEM((CH, NF), jnp.float32),
        pltpu.VMEM((B_PER_W,), jnp.float32),
        pltpu.SemaphoreType.DMA,
        pltpu.SemaphoreType.DMA,
    ],
)
def _mf_kernel(user_hbm, business_hbm, uf_hbm, bf_hbm, out_hbm,
               uidx, bidx, ubuf0, ubuf1, bbuf0, bbuf1, outv, sem0, sem1):
    wid = lax.axis_index("s") * _NC + lax.axis_index("c")
    base = wid * B_PER_W

    for j in range(NCHUNK):
        pltpu.sync_copy(user_hbm.at[pl.ds(base + j * CH, CH)], uidx.at[j])
        pltpu.sync_copy(business_hbm.at[pl.ds(base + j * CH, CH)], bidx.at[j])

    ubufs = (ubuf0, ubuf1)
    bbufs = (bbuf0, bbuf1)
    sems = (sem0, sem1)
    lanes = lax.iota(jnp.int32, _L)

    def start_gather(j):
        s = sems[j % 2]
        cu = pltpu.async_copy(uf_hbm.at[uidx.at[j]], ubufs[j % 2], s)
        cb = pltpu.async_copy(bf_hbm.at[bidx.at[j]], bbufs[j % 2], s)
        return cu, cb

    pending = start_gather(0)
    for j in range(NCHUNK):
        nxt = start_gather(j + 1) if j + 1 < NCHUNK else None
        pending[0].wait()
        pending[1].wait()
        ub = ubufs[j % 2]
        bb = bbufs[j % 2]

        def group_body(g, _):
            row = g * _L + lanes

            def d_body(d, acc):
                col = lax.bitwise_and(lanes + d, NF - 1)
                uvals = plsc.load_gather(ub, [row, col])
                bvals = plsc.load_gather(bb, [row, col])
                return acc + uvals * bvals

            acc = lax.fori_loop(0, NF, d_body, jnp.zeros((_L,), jnp.float32))
            outv[pl.ds(j * CH + g * _L, _L)] = acc
            return 0

        lax.fori_loop(0, CH // _L, group_body, 0)
        pending = nxt

    pltpu.sync_copy(outv, out_hbm.at[pl.ds(base, B_PER_W)])


def kernel(user, business, user_factors, business_factors):
    return _mf_kernel(user, business, user_factors, business_factors)
